# baseline (speedup 1.0000x reference)
import functools

import jax
import jax.numpy as jnp
from jax import lax
from jax.experimental import pallas as pl
from jax.experimental.pallas import tpu as pltpu
from jax.experimental.pallas import tpu_sc as plsc

_NVS = [10242, 2562, 642, 162, 42, 12]
_LANE = 16
_NW = 32

def _cdiv(a, b):
    return -(-a // b)


def _zmat(x, W):
    n, ci = x.shape
    co = W.shape[0]
    if ci < 128:
        Wt = W.reshape(co, 7, ci).transpose(1, 2, 0)

        def body_t(x_ref, w_ref, z_ref):
            xv = x_ref[...]
            for j in range(7):
                z_ref[j] = lax.dot_general(
                    xv, w_ref[j], (((1,), (0,)), ((), ())),
                    preferred_element_type=jnp.float32)

        bn = 2048 if n > 4096 else n
        return pl.pallas_call(
            body_t,
            grid=(_cdiv(n, bn),),
            in_specs=[pl.BlockSpec((bn, ci), lambda i: (i, 0)),
                      pl.BlockSpec((7, ci, co), lambda i: (0, 0, 0))],
            out_specs=pl.BlockSpec((7, bn, co), lambda i: (0, i, 0)),
            out_shape=jax.ShapeDtypeStruct((7, n, co), jnp.float32),
        )(x, Wt)

    def body(x_ref, w_ref, z_ref):
        xv = x_ref[...]
        for j in range(7):
            wj = w_ref[:, j * ci:(j + 1) * ci]
            z_ref[j] = lax.dot_general(
                xv, wj, (((1,), (1,)), ((), ())),
                preferred_element_type=jnp.float32)

    return pl.pallas_call(
        body,
        out_shape=jax.ShapeDtypeStruct((7, n, co), jnp.float32),
    )(x, W)


def _geom(n, co):
    V = _cdiv(_cdiv(n, _NW), 8) * 8
    CR = max(8, min(128, (230 * 1024) // (7 * co * 4) // 8 * 8, V))
    NCH = _cdiv(V, CR)
    VP = NCH * CR
    return V, CR, NCH, VP


def _sc_gather_sum(tab, nog, n, co, scale_inv7):
    V, CR, NCH, VP = _geom(n, co)
    nc = co // _LANE
    mesh = plsc.VectorSubcoreMesh(core_axis_name="c", subcore_axis_name="s")

    @functools.partial(
        pl.kernel,
        mesh=mesh,
        compiler_params=pltpu.CompilerParams(use_tc_tiling_on_sc=False),
        out_type=jax.ShapeDtypeStruct((_NW * V, co), jnp.float32),
        scratch_types=[
            pltpu.VMEM((VP, co), jnp.float32),
            pltpu.VMEM((7, CR, co), jnp.float32),
            pltpu.VMEM((NCH, 7, CR), jnp.int32),
            pltpu.SemaphoreType.DMA,
        ],
    )
    def k(tab_h, nog_h, out_h, y_v, g7_v, idx_v, sem):
        wid = lax.axis_index("s") * 2 + lax.axis_index("c")
        v0 = wid * V
        scale = 1.0 / 7.0
        pltpu.sync_copy(nog_h.at[wid], idx_v)

        for ch in range(NCH):
            cps = [pltpu.async_copy(
                tab_h.at[idx_v.at[ch, j]],
                g7_v.at[j], sem) for j in range(7)]
            for cp in cps:
                cp.wait()

            def acc_body(r, _):
                def acc_cc(cc, _2):
                    dsl = pl.ds(cc * _LANE, _LANE)
                    acc = g7_v[0, r, dsl]
                    for j in range(1, 7):
                        acc = acc + g7_v[j, r, dsl]
                    if scale_inv7:
                        acc = acc * scale
                    y_v[ch * CR + r, dsl] = acc
                    return 0
                return lax.fori_loop(0, nc, acc_cc, 0)
            lax.fori_loop(0, CR, acc_body, 0)

        pltpu.sync_copy(y_v.at[pl.ds(0, V)], out_h.at[pl.ds(v0, V)])

    return k(tab, nog)


def _tc_bn_lrelu(y, b, g, be, n):
    def body(y_ref, b_ref, g_ref, be_ref, o_ref):
        yv = y_ref[...] + b_ref[...]
        m = jnp.mean(yv, axis=0, keepdims=True)
        v = jnp.mean((yv - m) ** 2, axis=0, keepdims=True)
        o = (yv - m) * (g_ref[...] * lax.rsqrt(v + 1e-5)) + be_ref[...]
        o_ref[...] = jnp.where(o >= 0, o, 0.1 * o)

    co = y.shape[1]
    return pl.pallas_call(
        body,
        out_shape=jax.ShapeDtypeStruct((n, co), jnp.float32),
    )(y[:n], b.reshape(1, co), g.reshape(1, co), be.reshape(1, co))


def _tc_head(y, Wf, bf):
    def body(y_ref, wf_ref, bf_ref, o_ref):
        m = jnp.mean(y_ref[...], axis=0, keepdims=True)
        o_ref[0, 0] = jnp.sum(m * wf_ref[...]) + bf_ref[0, 0]

    return pl.pallas_call(
        body,
        out_specs=pl.BlockSpec(memory_space=pltpu.SMEM),
        out_shape=jax.ShapeDtypeStruct((1, 1), jnp.float32),
    )(y, Wf, bf.reshape(1, 1))


def _tile_grouped_idx(no, n, co, row_offset_scale):
    V, CR, NCH, VP = _geom(n, co)
    nop = jnp.pad(no.reshape(n, 7), ((0, _NW * V - n), (0, 0)))
    nop = nop.reshape(_NW, V, 7).transpose(0, 2, 1)
    nop = jnp.pad(nop, ((0, 0), (0, 0), (0, VP - V)))
    off = (jnp.arange(7, dtype=jnp.int32) * row_offset_scale)[None, :, None]
    nop = nop + off
    return nop.reshape(_NW, 7, NCH, CR).transpose(0, 2, 1, 3)


def _conv_layer(x, no, W, b, g, be):
    n = x.shape[0]
    co = W.shape[0]
    z = _zmat(x, W).reshape(7 * n, co)
    nog = _tile_grouped_idx(no, n, co, n)
    y = _sc_gather_sum(z, nog, n, co, False)
    return _tc_bn_lrelu(y, b, g, be, n)


def kernel(x, no0, no1, no2, no3, no4, no5,
           W0, b0, g0, be0, W1, b1, g1, be1, W2, b2, g2, be2,
           W3, b3, g3, be3, W4, b4, g4, be4, W5, b5, g5, be5,
           W6, b6, g6, be6, W7, b7, g7, be7, W8, b8, g8, be8,
           W9, b9, g9, be9, W10, b10, g10, be10, W11, b11, g11, be11,
           W12, b12, g12, be12, Wf, bf):
    nos = [no0, no1, no2, no3, no4, no5]
    Ws = [W0, W1, W2, W3, W4, W5, W6, W7, W8, W9, W10, W11, W12]
    bs = [b0, b1, b2, b3, b4, b5, b6, b7, b8, b9, b10, b11, b12]
    gs = [g0, g1, g2, g3, g4, g5, g6, g7, g8, g9, g10, g11, g12]
    bes = [be0, be1, be2, be3, be4, be5, be6, be7, be8, be9, be10, be11, be12]

    li = 0
    for _ in range(3):
        x = _conv_layer(x, nos[0], Ws[li], bs[li], gs[li], bes[li])
        li += 1
    for l in range(1, 6):
        m = _NVS[l]
        nog = _tile_grouped_idx(nos[l - 1][:7 * m], m, x.shape[1], 0)
        x = _sc_gather_sum(x, nog, m, x.shape[1], True)[:m]
        for _ in range(2):
            x = _conv_layer(x, nos[l], Ws[li], bs[li], gs[li], bes[li])
            li += 1
    return _tc_head(x, Wf, bf)


# speedup vs baseline: 1.1386x; 1.1386x over previous
import functools

import jax
import jax.numpy as jnp
from jax import lax
from jax.experimental import pallas as pl
from jax.experimental.pallas import tpu as pltpu
from jax.experimental.pallas import tpu_sc as plsc

_NVS = [10242, 2562, 642, 162, 42, 12]
_LANE = 16
_NW = 16

def _cdiv(a, b):
    return -(-a // b)


def _zmat(x, W):
    n, ci = x.shape
    co = W.shape[0]
    if ci < 128:
        Wt = W.reshape(co, 7, ci).transpose(1, 2, 0)

        def body_t(x_ref, w_ref, z_ref):
            xv = x_ref[...]
            for j in range(7):
                z_ref[j] = lax.dot_general(
                    xv, w_ref[j], (((1,), (0,)), ((), ())),
                    preferred_element_type=jnp.float32)

        bn = 2048 if n > 4096 else n
        return pl.pallas_call(
            body_t,
            grid=(_cdiv(n, bn),),
            in_specs=[pl.BlockSpec((bn, ci), lambda i: (i, 0)),
                      pl.BlockSpec((7, ci, co), lambda i: (0, 0, 0))],
            out_specs=pl.BlockSpec((7, bn, co), lambda i: (0, i, 0)),
            out_shape=jax.ShapeDtypeStruct((7, n, co), jnp.float32),
        )(x, Wt)

    def body(x_ref, w_ref, z_ref):
        xv = x_ref[...]
        for j in range(7):
            wj = w_ref[:, j * ci:(j + 1) * ci]
            z_ref[j] = lax.dot_general(
                xv, wj, (((1,), (1,)), ((), ())),
                preferred_element_type=jnp.float32)

    return pl.pallas_call(
        body,
        out_shape=jax.ShapeDtypeStruct((7, n, co), jnp.float32),
    )(x, W)


def _geom(n, co):
    V = _cdiv(_cdiv(n, _NW), 8) * 8
    CR = max(8, min(128, (230 * 1024) // (7 * co * 4) // 8 * 8, V))
    NCH = _cdiv(V, CR)
    VP = NCH * CR
    return V, CR, NCH, VP


def _sc_gather_sum(tab, nog, n, co, scale_inv7):
    V, CR, NCH, VP = _geom(n, co)
    nc = co // _LANE
    mesh = plsc.VectorSubcoreMesh(
        core_axis_name="c", subcore_axis_name="s", num_cores=1)

    @functools.partial(
        pl.kernel,
        mesh=mesh,
        compiler_params=pltpu.CompilerParams(use_tc_tiling_on_sc=False),
        out_type=jax.ShapeDtypeStruct((_NW * V, co), jnp.float32),
        scratch_types=[
            pltpu.VMEM((VP, co), jnp.float32),
            pltpu.VMEM((7, CR, co), jnp.float32),
            pltpu.VMEM((7, CR), jnp.int32),
            pltpu.SemaphoreType.DMA,
        ],
    )
    def k(tab_h, nog_h, out_h, y_v, g7_v, idx_v, sem):
        wid = lax.axis_index("s")
        v0 = wid * V
        scale = 1.0 / 7.0
        NWV = _NW * V

        for ch in range(NCH):
            for j in range(7):
                pltpu.sync_copy(
                    nog_h.at[pl.ds(j * NWV + v0 + ch * CR, CR)],
                    idx_v.at[j])
            cps = [pltpu.async_copy(tab_h.at[idx_v.at[j]], g7_v.at[j], sem)
                   for j in range(7)]
            for cp in cps:
                cp.wait()

            def acc_body(r, _):
                def acc_cc(cc, _2):
                    dsl = pl.ds(cc * _LANE, _LANE)
                    acc = g7_v[0, r, dsl]
                    for j in range(1, 7):
                        acc = acc + g7_v[j, r, dsl]
                    if scale_inv7:
                        acc = acc * scale
                    y_v[ch * CR + r, dsl] = acc
                    return 0
                return lax.fori_loop(0, nc, acc_cc, 0)
            lax.fori_loop(0, CR, acc_body, 0)

        pltpu.sync_copy(y_v.at[pl.ds(0, V)], out_h.at[pl.ds(v0, V)])

    return k(tab, nog)


def _tc_bn_lrelu(y, b, g, be, n):
    def body(y_ref, b_ref, g_ref, be_ref, o_ref):
        yv = y_ref[...] + b_ref[...]
        m = jnp.mean(yv, axis=0, keepdims=True)
        v = jnp.mean((yv - m) ** 2, axis=0, keepdims=True)
        o = (yv - m) * (g_ref[...] * lax.rsqrt(v + 1e-5)) + be_ref[...]
        o_ref[...] = jnp.where(o >= 0, o, 0.1 * o)

    co = y.shape[1]
    return pl.pallas_call(
        body,
        out_shape=jax.ShapeDtypeStruct((n, co), jnp.float32),
    )(y[:n], b.reshape(1, co), g.reshape(1, co), be.reshape(1, co))


def _tc_head(y, Wf, bf):
    def body(y_ref, wf_ref, bf_ref, o_ref):
        m = jnp.mean(y_ref[...], axis=0, keepdims=True)
        o_ref[0, 0] = jnp.sum(m * wf_ref[...]) + bf_ref[0, 0]

    return pl.pallas_call(
        body,
        out_specs=pl.BlockSpec(memory_space=pltpu.SMEM),
        out_shape=jax.ShapeDtypeStruct((1, 1), jnp.float32),
    )(y, Wf, bf.reshape(1, 1))


def _tile_grouped_idx(no, n, co, row_offset_scale):
    V = _geom(n, co)[0]
    NWV = _NW * V
    nop = jnp.pad(no.reshape(n, 7), ((0, NWV - n), (0, 0)))
    off = (jnp.arange(7, dtype=jnp.int32) * row_offset_scale)[:, None]
    return (nop.T + off).reshape(-1)


def _conv_layer(x, no, W, b, g, be):
    n = x.shape[0]
    co = W.shape[0]
    z = _zmat(x, W).reshape(7 * n, co)
    nog = _tile_grouped_idx(no, n, co, n)
    y = _sc_gather_sum(z, nog, n, co, False)
    return _tc_bn_lrelu(y, b, g, be, n)


def kernel(x, no0, no1, no2, no3, no4, no5,
           W0, b0, g0, be0, W1, b1, g1, be1, W2, b2, g2, be2,
           W3, b3, g3, be3, W4, b4, g4, be4, W5, b5, g5, be5,
           W6, b6, g6, be6, W7, b7, g7, be7, W8, b8, g8, be8,
           W9, b9, g9, be9, W10, b10, g10, be10, W11, b11, g11, be11,
           W12, b12, g12, be12, Wf, bf):
    nos = [no0, no1, no2, no3, no4, no5]
    Ws = [W0, W1, W2, W3, W4, W5, W6, W7, W8, W9, W10, W11, W12]
    bs = [b0, b1, b2, b3, b4, b5, b6, b7, b8, b9, b10, b11, b12]
    gs = [g0, g1, g2, g3, g4, g5, g6, g7, g8, g9, g10, g11, g12]
    bes = [be0, be1, be2, be3, be4, be5, be6, be7, be8, be9, be10, be11, be12]

    li = 0
    for _ in range(3):
        x = _conv_layer(x, nos[0], Ws[li], bs[li], gs[li], bes[li])
        li += 1
    for l in range(1, 6):
        m = _NVS[l]
        nog = _tile_grouped_idx(nos[l - 1][:7 * m], m, x.shape[1], 0)
        x = _sc_gather_sum(x, nog, m, x.shape[1], True)[:m]
        for _ in range(2):
            x = _conv_layer(x, nos[l], Ws[li], bs[li], gs[li], bes[li])
            li += 1
    return _tc_head(x, Wf, bf)


# speedup vs baseline: 1.2355x; 1.0851x over previous
import functools

import jax
import jax.numpy as jnp
from jax import lax
from jax.experimental import pallas as pl
from jax.experimental.pallas import tpu as pltpu
from jax.experimental.pallas import tpu_sc as plsc

_NVS = [10242, 2562, 642, 162, 42, 12]
_LANE = 16
_NW = 16

def _cdiv(a, b):
    return -(-a // b)


def _zmat(x, W):
    n, ci = x.shape
    co = W.shape[0]
    if ci < 128:
        Wt = W.reshape(co, 7, ci).transpose(1, 2, 0)

        def body_t(x_ref, w_ref, z_ref):
            xv = x_ref[...]
            for j in range(7):
                z_ref[j] = lax.dot_general(
                    xv, w_ref[j], (((1,), (0,)), ((), ())),
                    preferred_element_type=jnp.float32)

        bn = 2048 if n > 4096 else n
        return pl.pallas_call(
            body_t,
            grid=(_cdiv(n, bn),),
            in_specs=[pl.BlockSpec((bn, ci), lambda i: (i, 0)),
                      pl.BlockSpec((7, ci, co), lambda i: (0, 0, 0))],
            out_specs=pl.BlockSpec((7, bn, co), lambda i: (0, i, 0)),
            out_shape=jax.ShapeDtypeStruct((7, n, co), jnp.float32),
        )(x, Wt)

    def body(x_ref, w_ref, z_ref):
        xv = x_ref[...]
        for j in range(7):
            wj = w_ref[:, j * ci:(j + 1) * ci]
            z_ref[j] = lax.dot_general(
                xv, wj, (((1,), (1,)), ((), ())),
                preferred_element_type=jnp.float32)

    return pl.pallas_call(
        body,
        out_shape=jax.ShapeDtypeStruct((7, n, co), jnp.float32),
    )(x, W)


def _geom(n, co):
    V = _cdiv(_cdiv(n, _NW), 8) * 8
    CR = max(8, min(128, (100 * 1024) // (7 * co * 4) // 8 * 8, V))
    NCH = _cdiv(V, CR)
    VP = NCH * CR
    return V, CR, NCH, VP


def _sc_gather_sum(tab, nog, n, co, scale_inv7):
    V, CR, NCH, VP = _geom(n, co)
    nc = co // _LANE
    mesh = plsc.VectorSubcoreMesh(
        core_axis_name="c", subcore_axis_name="s", num_cores=1)

    @functools.partial(
        pl.kernel,
        mesh=mesh,
        compiler_params=pltpu.CompilerParams(use_tc_tiling_on_sc=False),
        out_type=jax.ShapeDtypeStruct((_NW * V, co), jnp.float32),
        scratch_types=[
            pltpu.VMEM((VP, co), jnp.float32),
            pltpu.VMEM((2, 7, CR, co), jnp.float32),
            pltpu.VMEM((2, 7, CR), jnp.int32),
            pltpu.SemaphoreType.DMA,
            pltpu.SemaphoreType.DMA,
        ],
    )
    def k(tab_h, nog_h, out_h, y_v, g7_v, idx_v, sem0, sem1):
        wid = lax.axis_index("s")
        v0 = wid * V
        scale = 1.0 / 7.0
        NWV = _NW * V
        sems = (sem0, sem1)

        def do_idx(ch, slot):
            for j in range(7):
                pltpu.sync_copy(
                    nog_h.at[pl.ds(j * NWV + v0 + ch * CR, CR)],
                    idx_v.at[slot, j])

        def fire(ch, slot):
            return [pltpu.async_copy(tab_h.at[idx_v.at[slot, j]],
                                     g7_v.at[slot, j], sems[slot])
                    for j in range(7)]

        do_idx(0, 0)
        cps = fire(0, 0)
        for ch in range(NCH):
            slot = ch % 2
            nxt = None
            if ch + 1 < NCH:
                do_idx(ch + 1, 1 - slot)
                nxt = fire(ch + 1, 1 - slot)
            for cp in cps:
                cp.wait()

            def acc_body(r, _):
                def acc_cc(cc, _2):
                    dsl = pl.ds(cc * _LANE, _LANE)
                    acc = g7_v[slot, 0, r, dsl]
                    for j in range(1, 7):
                        acc = acc + g7_v[slot, j, r, dsl]
                    if scale_inv7:
                        acc = acc * scale
                    y_v[ch * CR + r, dsl] = acc
                    return 0
                return lax.fori_loop(0, nc, acc_cc, 0)
            lax.fori_loop(0, CR, acc_body, 0)
            cps = nxt

        pltpu.sync_copy(y_v.at[pl.ds(0, V)], out_h.at[pl.ds(v0, V)])

    return k(tab, nog)


def _tc_bn_lrelu(y, b, g, be, n):
    def body(y_ref, b_ref, g_ref, be_ref, o_ref):
        yv = y_ref[...] + b_ref[...]
        m = jnp.mean(yv, axis=0, keepdims=True)
        v = jnp.mean((yv - m) ** 2, axis=0, keepdims=True)
        o = (yv - m) * (g_ref[...] * lax.rsqrt(v + 1e-5)) + be_ref[...]
        o_ref[...] = jnp.where(o >= 0, o, 0.1 * o)

    co = y.shape[1]
    return pl.pallas_call(
        body,
        out_shape=jax.ShapeDtypeStruct((n, co), jnp.float32),
    )(y[:n], b.reshape(1, co), g.reshape(1, co), be.reshape(1, co))


def _tc_head(y, Wf, bf):
    def body(y_ref, wf_ref, bf_ref, o_ref):
        m = jnp.mean(y_ref[...], axis=0, keepdims=True)
        o_ref[0, 0] = jnp.sum(m * wf_ref[...]) + bf_ref[0, 0]

    return pl.pallas_call(
        body,
        out_specs=pl.BlockSpec(memory_space=pltpu.SMEM),
        out_shape=jax.ShapeDtypeStruct((1, 1), jnp.float32),
    )(y, Wf, bf.reshape(1, 1))


def _tile_grouped_idx(no, n, co, row_offset_scale):
    V = _geom(n, co)[0]
    NWV = _NW * V
    nop = jnp.pad(no.reshape(n, 7), ((0, NWV - n), (0, 0)))
    off = (jnp.arange(7, dtype=jnp.int32) * row_offset_scale)[:, None]
    return (nop.T + off).reshape(-1)


def _conv_layer(x, no, W, b, g, be):
    n = x.shape[0]
    co = W.shape[0]
    z = _zmat(x, W).reshape(7 * n, co)
    nog = _tile_grouped_idx(no, n, co, n)
    y = _sc_gather_sum(z, nog, n, co, False)
    return _tc_bn_lrelu(y, b, g, be, n)


def kernel(x, no0, no1, no2, no3, no4, no5,
           W0, b0, g0, be0, W1, b1, g1, be1, W2, b2, g2, be2,
           W3, b3, g3, be3, W4, b4, g4, be4, W5, b5, g5, be5,
           W6, b6, g6, be6, W7, b7, g7, be7, W8, b8, g8, be8,
           W9, b9, g9, be9, W10, b10, g10, be10, W11, b11, g11, be11,
           W12, b12, g12, be12, Wf, bf):
    nos = [no0, no1, no2, no3, no4, no5]
    Ws = [W0, W1, W2, W3, W4, W5, W6, W7, W8, W9, W10, W11, W12]
    bs = [b0, b1, b2, b3, b4, b5, b6, b7, b8, b9, b10, b11, b12]
    gs = [g0, g1, g2, g3, g4, g5, g6, g7, g8, g9, g10, g11, g12]
    bes = [be0, be1, be2, be3, be4, be5, be6, be7, be8, be9, be10, be11, be12]

    li = 0
    for _ in range(3):
        x = _conv_layer(x, nos[0], Ws[li], bs[li], gs[li], bes[li])
        li += 1
    for l in range(1, 6):
        m = _NVS[l]
        nog = _tile_grouped_idx(nos[l - 1][:7 * m], m, x.shape[1], 0)
        x = _sc_gather_sum(x, nog, m, x.shape[1], True)[:m]
        for _ in range(2):
            x = _conv_layer(x, nos[l], Ws[li], bs[li], gs[li], bes[li])
            li += 1
    return _tc_head(x, Wf, bf)
